# baseline (speedup 1.0000x reference)
import jax
import jax.numpy as jnp
from jax.experimental import pallas as pl
from jax.experimental.pallas import tpu as pltpu

_BLOCK_ROWS = 128


def _physics_mask_kernel(
    lg_ref, z_ref, a_ref, o_ref, zcache_ref, zmax_ref, zsum_ref, asum_ref
):
    i = pl.program_id(0)
    nb = pl.num_programs(0) // 2

    @pl.when(i < nb)
    def _reduce_phase():
        z = z_ref[...]
        a = a_ref[...]
        zcache_ref[pl.ds(i * _BLOCK_ROWS, _BLOCK_ROWS), :] = z.astype(jnp.bfloat16)
        bmax = jnp.max(z)
        bzsum = jnp.sum(jnp.abs(z))
        basum = jnp.sum(jnp.abs(a))

        @pl.when(i == 0)
        def _():
            zmax_ref[0, 0] = bmax
            zsum_ref[0, 0] = bzsum
            asum_ref[0, 0] = basum

        @pl.when(i != 0)
        def _():
            zmax_ref[0, 0] = jnp.maximum(zmax_ref[0, 0], bmax)
            zsum_ref[0, 0] = zsum_ref[0, 0] + bzsum
            asum_ref[0, 0] = asum_ref[0, 0] + basum

    @pl.when(i >= nb)
    def _emit_phase():
        j = i - nb
        n_elems = _BLOCK_ROWS * o_ref.shape[1] * nb
        gamma = jnp.exp(lg_ref[0, 0])
        z_max = zmax_ref[0, 0] + 1e-8
        inv_n = 1.0 / jnp.float32(n_elems)
        attn_scale = jnp.maximum(asum_ref[0, 0] * inv_n, 1e-4)
        mask_scale = jnp.maximum(
            gamma * zsum_ref[0, 0] * inv_n / jnp.abs(z_max), 1e-4
        )
        coeff = (-gamma / z_max) * (attn_scale / mask_scale)
        zc = zcache_ref[pl.ds(j * _BLOCK_ROWS, _BLOCK_ROWS), :].astype(jnp.float32)
        o_ref[...] = zc * coeff


def kernel(impedance_matrix, attn_logits, log_gamma):
    n, m = impedance_matrix.shape
    nb = n // _BLOCK_ROWS
    block = (_BLOCK_ROWS, m)

    lg = jnp.reshape(log_gamma.astype(jnp.float32), (1, 1))

    def in_map(i):
        return (jnp.minimum(i, nb - 1), 0)

    def out_map(i):
        return (jnp.maximum(i - nb, 0), 0)

    out = pl.pallas_call(
        _physics_mask_kernel,
        grid=(2 * nb,),
        in_specs=[
            pl.BlockSpec((1, 1), lambda i: (0, 0), memory_space=pltpu.SMEM),
            pl.BlockSpec(block, in_map),
            pl.BlockSpec(block, in_map),
        ],
        out_specs=pl.BlockSpec(block, out_map),
        out_shape=jax.ShapeDtypeStruct((n, m), jnp.float32),
        scratch_shapes=[
            pltpu.VMEM((n, m), jnp.bfloat16),
            pltpu.SMEM((1, 1), jnp.float32),
            pltpu.SMEM((1, 1), jnp.float32),
            pltpu.SMEM((1, 1), jnp.float32),
        ],
        compiler_params=pltpu.CompilerParams(
            dimension_semantics=("arbitrary",),
        ),
    )(lg, impedance_matrix, attn_logits)

    return out


# speedup vs baseline: 1.1960x; 1.1960x over previous
import jax
import jax.numpy as jnp
from jax.experimental import pallas as pl
from jax.experimental.pallas import tpu as pltpu

_BLOCK_ROWS = 256


def _physics_mask_kernel(
    lg_ref, z_ref, a_ref, o_ref, zcache_ref, zmax_ref, zsum_ref, asum_ref
):
    i = pl.program_id(0)
    nb = pl.num_programs(0) // 2

    @pl.when(i < nb)
    def _reduce_phase():
        z = z_ref[...]
        a = a_ref[...]
        zcache_ref[pl.ds(i * _BLOCK_ROWS, _BLOCK_ROWS), :] = z.astype(jnp.bfloat16)
        bmax = jnp.max(z)
        bzsum = jnp.sum(jnp.abs(z))
        basum = jnp.sum(jnp.abs(a))

        @pl.when(i == 0)
        def _():
            zmax_ref[0, 0] = bmax
            zsum_ref[0, 0] = bzsum
            asum_ref[0, 0] = basum

        @pl.when(i != 0)
        def _():
            zmax_ref[0, 0] = jnp.maximum(zmax_ref[0, 0], bmax)
            zsum_ref[0, 0] = zsum_ref[0, 0] + bzsum
            asum_ref[0, 0] = asum_ref[0, 0] + basum

    @pl.when(i >= nb)
    def _emit_phase():
        j = i - nb
        n_elems = _BLOCK_ROWS * o_ref.shape[1] * nb
        gamma = jnp.exp(lg_ref[0, 0])
        z_max = zmax_ref[0, 0] + 1e-8
        inv_n = 1.0 / jnp.float32(n_elems)
        attn_scale = jnp.maximum(asum_ref[0, 0] * inv_n, 1e-4)
        mask_scale = jnp.maximum(
            gamma * zsum_ref[0, 0] * inv_n / jnp.abs(z_max), 1e-4
        )
        coeff = (-gamma / z_max) * (attn_scale / mask_scale)
        zc = zcache_ref[pl.ds(j * _BLOCK_ROWS, _BLOCK_ROWS), :].astype(jnp.float32)
        o_ref[...] = zc * coeff


def kernel(impedance_matrix, attn_logits, log_gamma):
    n, m = impedance_matrix.shape
    nb = n // _BLOCK_ROWS
    block = (_BLOCK_ROWS, m)

    lg = jnp.reshape(log_gamma.astype(jnp.float32), (1, 1))

    def in_map(i):
        return (jnp.minimum(i, nb - 1), 0)

    def out_map(i):
        return (jnp.maximum(i - nb, 0), 0)

    out = pl.pallas_call(
        _physics_mask_kernel,
        grid=(2 * nb,),
        in_specs=[
            pl.BlockSpec((1, 1), lambda i: (0, 0), memory_space=pltpu.SMEM),
            pl.BlockSpec(block, in_map),
            pl.BlockSpec(block, in_map),
        ],
        out_specs=pl.BlockSpec(block, out_map),
        out_shape=jax.ShapeDtypeStruct((n, m), jnp.float32),
        scratch_shapes=[
            pltpu.VMEM((n, m), jnp.bfloat16),
            pltpu.SMEM((1, 1), jnp.float32),
            pltpu.SMEM((1, 1), jnp.float32),
            pltpu.SMEM((1, 1), jnp.float32),
        ],
        compiler_params=pltpu.CompilerParams(
            dimension_semantics=("arbitrary",),
        ),
    )(lg, impedance_matrix, attn_logits)

    return out
